# baseline (speedup 1.0000x reference)
import functools

import jax
import jax.numpy as jnp
from jax import lax
from jax.experimental import pallas as pl
from jax.experimental.pallas import tpu as pltpu
from jax.experimental.pallas import tpu_sc as plsc

NCOND = 1000
ED = 64
L = 16
D, WD, H = 4, 8, 8
BLANE = 128


def _make_lookup(B: int):
    info = plsc.get_sparse_core_info()
    nc, ns = info.num_cores, info.num_subcores
    nw = nc * ns
    epw = ED // nw
    nbt = B // BLANE
    assert ED % nw == 0 and B % BLANE == 0
    mesh = plsc.VectorSubcoreMesh(core_axis_name="c", subcore_axis_name="s")

    @functools.partial(
        pl.kernel,
        mesh=mesh,
        compiler_params=pltpu.CompilerParams(needs_layout_passes=False),
        out_type=jax.ShapeDtypeStruct((ED, D, WD, nbt, H, BLANE), jnp.float32),
        scratch_types=[
            pltpu.VMEM((epw * NCOND + L, ), jnp.float32),
            pltpu.VMEM((B,), jnp.int32),
            pltpu.VMEM((ED,), jnp.float32),
            pltpu.VMEM((L,), jnp.float32),
            pltpu.VMEM((epw, nbt, H, BLANE), jnp.float32),
            pltpu.SemaphoreType.DMA,
            pltpu.SemaphoreType.DMA,
        ],
    )
    def lookup(w_hbm, idx_hbm, b_hbm, one_hbm, out_hbm,
               w_v, idx_v, b_v, one_v, blk, sem0, sem1):
        wid = lax.axis_index("s") * nc + lax.axis_index("c")
        e0 = wid * epw

        cp_w = pltpu.make_async_copy(
            w_hbm.at[pl.ds(e0 * NCOND, epw * NCOND)],
            w_v.at[pl.ds(0, epw * NCOND)], sem0)
        cp_i = pltpu.make_async_copy(idx_hbm, idx_v, sem0)
        cp_b = pltpu.make_async_copy(b_hbm, b_v, sem0)
        cp_o = pltpu.make_async_copy(one_hbm, one_v, sem0)
        for cp in (cp_w, cp_i, cp_b, cp_o):
            cp.start()
        for cp in (cp_w, cp_i, cp_b, cp_o):
            cp.wait()

        ntail = NCOND % L
        nfull = NCOND // L
        lane = lax.iota(jnp.int32, L)
        one = one_v[pl.ds(0, L)]

        def build_block(le):
            rowbase = le * NCOND
            bias = plsc.load_gather(b_v, [jnp.full((L,), e0 + le, jnp.int32)])

            def sbody(c, acc):
                return acc + w_v[pl.ds(rowbase + L * c, L)]
            acc = lax.fori_loop(
                0, nfull, sbody, jnp.zeros((L,), jnp.float32))
            tail = w_v[pl.ds(rowbase + L * nfull, L)]
            acc = acc + jnp.where(lane < ntail, tail, 0.0)
            total = jnp.sum(acc)
            w_e0 = w_v[pl.ds(rowbase, L)][0]
            mh_vec = jnp.full((L,), total - w_e0, jnp.float32) + bias

            def cbody(c, carry):
                cvec = idx_v[pl.ds(L * c, L)]
                g = plsc.load_gather(
                    w_v, [jnp.minimum(cvec, NCOND - 1) + rowbase])
                val = jnp.where(cvec < NCOND, g + bias, mh_vec) * one
                k = c // (BLANE // L)
                lanepos = L * (c % (BLANE // L))
                for h in range(H):
                    blk[le, k, h, pl.ds(lanepos, L)] = val
                return carry
            lax.fori_loop(0, B // L, cbody, 0)

        build_block(0)
        build_block(1)
        for d in range(D):
            for w in range(WD):
                pltpu.make_async_copy(
                    blk, out_hbm.at[pl.ds(e0, epw), d, w],
                    sem0 if w % 2 == 0 else sem1).start()
        for d in range(D):
            for w in range(WD):
                pltpu.make_async_copy(
                    blk, out_hbm.at[pl.ds(e0, epw), d, w],
                    sem0 if w % 2 == 0 else sem1).wait()

    return lookup


def kernel(condition, spatial_shape, W, b):
    dims = jnp.asarray(spatial_shape)
    one = (dims[0] - D + dims[1] - WD + dims[2] - H + 1).astype(jnp.float32)
    B = condition.shape[0]
    idx = condition.astype(jnp.int32)
    one_arr = jnp.full((L,), one, jnp.float32)
    out6 = _make_lookup(B)(W.reshape(ED * NCOND), idx, b, one_arr)
    out5 = out6.transpose(3, 5, 0, 1, 2, 4).reshape(B, ED, D, WD, H)
    return out5


# speedup vs baseline: 1.0241x; 1.0241x over previous
import functools

import jax
import jax.numpy as jnp
from jax import lax
from jax.experimental import pallas as pl
from jax.experimental.pallas import tpu as pltpu
from jax.experimental.pallas import tpu_sc as plsc

NCOND = 1000
ED = 64
L = 16
D, WD, H = 4, 8, 8
BLANE = 128


def _make_lookup(B: int):
    info = plsc.get_sparse_core_info()
    nc, ns = info.num_cores, info.num_subcores
    nw = nc * ns
    epw = ED // nw
    nbt = B // BLANE
    assert ED % nw == 0 and B % BLANE == 0
    mesh = plsc.VectorSubcoreMesh(core_axis_name="c", subcore_axis_name="s")

    @functools.partial(
        pl.kernel,
        mesh=mesh,
        compiler_params=pltpu.CompilerParams(needs_layout_passes=False),
        out_type=jax.ShapeDtypeStruct((ED, D, WD, nbt, H, BLANE), jnp.float32),
        scratch_types=[
            pltpu.VMEM((epw * NCOND + L, ), jnp.float32),
            pltpu.VMEM((B,), jnp.int32),
            pltpu.VMEM((ED,), jnp.float32),
            pltpu.VMEM((L,), jnp.float32),
            pltpu.VMEM((nbt, H, BLANE), jnp.float32),
            pltpu.VMEM((nbt, H, BLANE), jnp.float32),
            pltpu.SemaphoreType.DMA,
            pltpu.SemaphoreType.DMA,
        ],
    )
    def lookup(w_hbm, idx_hbm, b_hbm, one_hbm, out_hbm,
               w_v, idx_v, b_v, one_v, blk0, blk1, sem0, sem1):
        wid = lax.axis_index("s") * nc + lax.axis_index("c")
        e0 = wid * epw

        cp_w = pltpu.make_async_copy(
            w_hbm.at[pl.ds(e0 * NCOND, epw * NCOND)],
            w_v.at[pl.ds(0, epw * NCOND)], sem0)
        cp_i = pltpu.make_async_copy(idx_hbm, idx_v, sem0)
        cp_b = pltpu.make_async_copy(b_hbm, b_v, sem0)
        cp_o = pltpu.make_async_copy(one_hbm, one_v, sem0)
        for cp in (cp_w, cp_i, cp_b, cp_o):
            cp.start()
        for cp in (cp_w, cp_i, cp_b, cp_o):
            cp.wait()

        ntail = NCOND % L
        nfull = NCOND // L
        lane = lax.iota(jnp.int32, L)
        one = one_v[pl.ds(0, L)]

        def build_block(le, blk):
            rowbase = le * NCOND
            bias = plsc.load_gather(b_v, [jnp.full((L,), e0 + le, jnp.int32)])

            def sbody(c, acc):
                return acc + w_v[pl.ds(rowbase + L * c, L)]
            acc = lax.fori_loop(
                0, nfull, sbody, jnp.zeros((L,), jnp.float32))
            tail = w_v[pl.ds(rowbase + L * nfull, L)]
            acc = acc + jnp.where(lane < ntail, tail, 0.0)
            total = jnp.sum(acc)
            w_e0 = w_v[pl.ds(rowbase, L)][0]
            mh_vec = jnp.full((L,), total - w_e0, jnp.float32) + bias

            def cbody(c, carry):
                cvec = idx_v[pl.ds(L * c, L)]
                g = plsc.load_gather(
                    w_v, [jnp.minimum(cvec, NCOND - 1) + rowbase])
                val = jnp.where(cvec < NCOND, g + bias, mh_vec) * one
                k = c // (BLANE // L)
                lanepos = L * (c % (BLANE // L))
                for h in range(H):
                    blk[k, h, pl.ds(lanepos, L)] = val
                return carry
            lax.fori_loop(0, B // L, cbody, 0)

        def fire(le, blk, sem):
            for d in range(D):
                for w in range(WD):
                    pltpu.make_async_copy(
                        blk, out_hbm.at[e0 + le, d, w], sem).start()

        def drain(le, blk, sem):
            for d in range(D):
                for w in range(WD):
                    pltpu.make_async_copy(
                        blk, out_hbm.at[e0 + le, d, w], sem).wait()

        build_block(0, blk0)
        fire(0, blk0, sem0)
        build_block(1, blk1)
        fire(1, blk1, sem1)
        drain(0, blk0, sem0)
        drain(1, blk1, sem1)

    return lookup


def kernel(condition, spatial_shape, W, b):
    dims = jnp.asarray(spatial_shape)
    one = (dims[0] - D + dims[1] - WD + dims[2] - H + 1).astype(jnp.float32)
    B = condition.shape[0]
    idx = condition.astype(jnp.int32)
    one_arr = jnp.full((L,), one, jnp.float32)
    out6 = _make_lookup(B)(W.reshape(ED * NCOND), idx, b, one_arr)
    out5 = out6.transpose(3, 5, 0, 1, 2, 4).reshape(B, ED, D, WD, H)
    return out5
